# Initial kernel scaffold; baseline (speedup 1.0000x reference)
import jax
import jax.numpy as jnp
from jax import lax
from jax.experimental import pallas as pl
from jax.experimental.pallas import tpu as pltpu
from jax.experimental.pallas import tpu_sc as plsc

_MARGIN = 0.3
_K = 8
_N = 512
_L = 16
_NC = 2
_NS = 16
_NW = _NC * _NS
_RPW = _N // _NW
_NCHUNK = _N // _L


def _merge16(best, chunk_sorted):
    lo = jnp.minimum(best, lax.rev(chunk_sorted, (0,)))
    return jnp.sort(lo)


def _sc_body(tbl_ref, tgt_ref, out_ref, t_v, rows_v, an_v, loss_v, sem):
    cid = lax.axis_index("c")
    sid = lax.axis_index("s")
    wid = sid * _NC + cid
    base = wid * _RPW

    pltpu.sync_copy(tgt_ref, t_v)
    idxv = t_v[pl.ds(base, _L)]
    pltpu.async_copy(tbl_ref.at[idxv], rows_v, sem).wait()

    iota = lax.iota(jnp.int32, _L)
    lane_lt8 = iota < _K

    def row_body(l, acc):
        ti = plsc.load_gather(t_v, [jnp.full((_L,), base + l, jnp.int32)])

        def mmx(c, carry):
            mx, mn = carry
            v = rows_v[l, pl.ds(c * _L, _L)]
            return jnp.maximum(mx, v), jnp.minimum(mn, v)

        v0 = rows_v[l, pl.ds(0, _L)]
        mx, mn = lax.fori_loop(1, _NCHUNK, mmx, (v0, v0))
        max1 = jnp.max(mx)
        min2 = jnp.min(mn)

        def topk(c, carry):
            b1, b2 = carry
            v = rows_v[l, pl.ds(c * _L, _L)]
            tc_ = t_v[pl.ds(c * _L, _L)]
            m = tc_ == ti
            m1 = jnp.where(m, v - max1, 0.0)
            m2n = jnp.where(m, 0.0, min2 - v)
            b1 = lax.cond(
                jnp.min(m1) < jnp.max(b1),
                lambda a, ch: _merge16(a, jnp.sort(ch)),
                lambda a, ch: a,
                b1, m1)
            b2 = lax.cond(
                jnp.min(m2n) < jnp.max(b2),
                lambda a, ch: _merge16(a, jnp.sort(ch)),
                lambda a, ch: a,
                b2, m2n)
            return b1, b2

        inf = jnp.full((_L,), jnp.inf, jnp.float32)
        b1, b2 = lax.fori_loop(0, _NCHUNK, topk, (inf, inf))

        g = plsc.load_gather(
            rows_v,
            [jnp.full((_L,), l, jnp.int32), jnp.full((_L,), base + l, jnp.int32)])
        ap = jnp.abs(g - (b1 + max1))
        an = (g - min2) + b2
        an_v[...] = an

        def pair(bi, a2):
            anb = plsc.load_gather(an_v, [jnp.full((_L,), bi, jnp.int32)])
            t = jnp.maximum(ap - anb + _MARGIN, 0.0)
            return a2 + jnp.where(lane_lt8, t, 0.0)

        return lax.fori_loop(0, _K, pair, acc)

    acc = lax.fori_loop(0, _RPW, row_body, jnp.zeros((_L,), jnp.float32))
    loss_v[...] = acc * (1.0 / (_N * _K * _K))
    pltpu.sync_copy(loss_v, out_ref.at[wid])


def _make_sc_kernel(interpret=False):
    return pl.kernel(
        _sc_body,
        out_type=jax.ShapeDtypeStruct((_NW, _L), jnp.float32),
        mesh=plsc.VectorSubcoreMesh(core_axis_name="c", subcore_axis_name="s"),
        scratch_types=[
            pltpu.VMEM((_N,), jnp.int32),
            pltpu.VMEM((_RPW, _N), jnp.float32),
            pltpu.VMEM((_L,), jnp.float32),
            pltpu.VMEM((_L,), jnp.float32),
            pltpu.SemaphoreType.DMA,
        ],
        interpret=interpret,
    )


def _sum_body(x_ref, o_ref):
    o_ref[0, 0] = jnp.sum(x_ref[...])


@jax.jit
def kernel(inputs, targets):
    inputs_t = inputs.T
    partial = _make_sc_kernel()(inputs_t, targets)
    loss = pl.pallas_call(
        _sum_body,
        out_shape=jax.ShapeDtypeStruct((1, 1), jnp.float32),
    )(partial)
    return loss[0, 0]


# speedup vs baseline: 3.8923x; 3.8923x over previous
import jax
import jax.numpy as jnp
from jax import lax
from jax.experimental import pallas as pl
from jax.experimental.pallas import tpu as pltpu
from jax.experimental.pallas import tpu_sc as plsc

_MARGIN = 0.3
_K = 8
_N = 512
_L = 16
_NC = 2
_NS = 16
_NW = _NC * _NS
_RPW = _N // _NW
_NCHUNK = _N // _L


def _merge16(best, chunk_sorted):
    lo = jnp.minimum(best, lax.rev(chunk_sorted, (0,)))
    return jnp.sort(lo)


def _sc_body(tbl_ref, tgt_ref, out_ref, t_v, rows_v, loss_v, sem):
    cid = lax.axis_index("c")
    sid = lax.axis_index("s")
    wid = sid * _NC + cid
    base = wid * _RPW

    pltpu.sync_copy(tgt_ref, t_v)
    idxv = t_v[pl.ds(base, _L)]
    pltpu.async_copy(tbl_ref.at[idxv], rows_v, sem).wait()

    iota = lax.iota(jnp.int32, _L)
    lane_lt8 = iota < _K

    def row_body(l, acc):
        lane_l = iota == l
        ti = jnp.sum(jnp.where(lane_l, idxv, 0))

        def mmx(c, carry):
            mx, mn = carry
            v = rows_v[l, pl.ds(c * _L, _L)]
            return jnp.maximum(mx, v), jnp.minimum(mn, v)

        v0 = rows_v[l, pl.ds(0, _L)]
        mx, mn = lax.fori_loop(1, _NCHUNK, mmx, (v0, v0))
        max1 = jnp.max(mx)
        min2 = jnp.min(mn)

        def topk(c, carry):
            b1, b2 = carry
            v = rows_v[l, pl.ds(c * _L, _L)]
            tc_ = t_v[pl.ds(c * _L, _L)]
            m = tc_ == ti
            m1 = jnp.where(m, v - max1, 0.0)
            m2n = jnp.where(m, 0.0, min2 - v)
            b1 = lax.cond(
                jnp.min(m1) < jnp.max(b1),
                lambda a, ch: _merge16(a, jnp.sort(ch)),
                lambda a, ch: a,
                b1, m1)
            b2 = lax.cond(
                jnp.min(m2n) < jnp.max(b2),
                lambda a, ch: _merge16(a, jnp.sort(ch)),
                lambda a, ch: a,
                b2, m2n)
            return b1, b2

        inf = jnp.full((_L,), jnp.inf, jnp.float32)
        b1, b2 = lax.fori_loop(0, _NCHUNK, topk, (inf, inf))

        g_vec = rows_v[l, pl.ds(base, _L)]
        g = jnp.sum(jnp.where(lane_l, g_vec, 0.0))
        ap = jnp.abs(g - (b1 + max1))
        an = (g - min2) + b2

        def pair(bi, a2):
            anb = jnp.sum(jnp.where(iota == bi, an, 0.0))
            t = jnp.maximum(ap - anb + _MARGIN, 0.0)
            return a2 + jnp.where(lane_lt8, t, 0.0)

        return lax.fori_loop(0, _K, pair, acc)

    acc = lax.fori_loop(0, _RPW, row_body, jnp.zeros((_L,), jnp.float32))
    loss_v[...] = acc * (1.0 / (_N * _K * _K))
    pltpu.sync_copy(loss_v, out_ref.at[wid])


def _make_sc_kernel(interpret=False):
    return pl.kernel(
        _sc_body,
        out_type=jax.ShapeDtypeStruct((_NW, _L), jnp.float32),
        mesh=plsc.VectorSubcoreMesh(
            core_axis_name="c", subcore_axis_name="s",
            num_cores=_NC, num_subcores=_NS),
        scratch_types=[
            pltpu.VMEM((_N,), jnp.int32),
            pltpu.VMEM((_RPW, _N), jnp.float32),
            pltpu.VMEM((_L,), jnp.float32),
            pltpu.SemaphoreType.DMA,
        ],
        compiler_params=pltpu.CompilerParams(needs_layout_passes=False),
        interpret=interpret,
    )


def _sum_body(x_ref, o_ref):
    o_ref[...] = jnp.full((1, 1), jnp.sum(x_ref[...]), jnp.float32)


@jax.jit
def kernel(inputs, targets):
    inputs_t = inputs.T
    partial = _make_sc_kernel()(inputs_t, targets)
    loss = pl.pallas_call(
        _sum_body,
        out_shape=jax.ShapeDtypeStruct((1, 1), jnp.float32),
    )(partial)
    return loss[0, 0]
